# baseline (speedup 1.0000x reference)
import functools

import jax
import jax.numpy as jnp
from jax import lax
from jax.experimental import pallas as pl
from jax.experimental.pallas import tpu as pltpu
from jax.experimental.pallas import tpu_sc as plsc

DIM = 1024
HEADS = 8
NUM_KEYS = 256
DIM_KEY = 128
TOPK = 8
KTOT = HEADS * TOPK
N_TOKENS = 2048
TB = 256
NEG = -3.0e38


def _top8(s, width):
    iota = lax.broadcasted_iota(jnp.int32, s.shape, 1).astype(jnp.float32)
    vs, ix = [], []
    for _ in range(8):
        m = jnp.max(s, axis=1, keepdims=True)
        pos = -jnp.max(jnp.where(s >= m, -iota, NEG), axis=1, keepdims=True)
        vs.append(m)
        ix.append(pos)
        s = jnp.where(iota == pos, NEG, s)
    return jnp.concatenate(vs, axis=1), jnp.concatenate(ix, axis=1)


def _routing_body(x_ref, wq_ref, kt_ref, eidx_ref, coeff_ref):
    xb = x_ref[...]
    for h in range(HEADS):
        svals, sidxs = [], []
        for p in range(2):
            col = (p * HEADS + h) * DIM_KEY
            q_ph = jnp.dot(xb, wq_ref[:, col:col + DIM_KEY],
                           preferred_element_type=jnp.float32)
            sim = jnp.dot(q_ph, kt_ref[p, h],
                          preferred_element_type=jnp.float32)
            v, i = _top8(sim, NUM_KEYS)
            svals.append(v)
            sidxs.append(i)
        sx, sy = svals
        ixx, ixy = sidxs
        alls = jnp.concatenate([sx[:, i:i + 1] + sy for i in range(8)], axis=1)
        alli = jnp.concatenate(
            [ixx[:, i:i + 1] * NUM_KEYS + ixy for i in range(8)], axis=1)
        iota = lax.broadcasted_iota(jnp.int32, alls.shape, 1).astype(jnp.float32)
        s = alls
        vs, es = [], []
        for _ in range(8):
            m = jnp.max(s, axis=1, keepdims=True)
            pos = -jnp.max(jnp.where(s >= m, -iota, NEG), axis=1,
                           keepdims=True)
            sel = iota == pos
            e = jnp.max(jnp.where(sel, alli, NEG), axis=1, keepdims=True)
            vs.append(m)
            es.append(e)
            s = jnp.where(sel, NEG, s)
        v8 = jnp.concatenate(vs, axis=1)
        e8 = jnp.concatenate(es, axis=1).astype(jnp.int32)
        m8 = jnp.max(v8, axis=1, keepdims=True)
        ex = jnp.exp(v8 - m8)
        cf = ex / jnp.sum(ex, axis=1, keepdims=True)
        eidx_ref[:, h * 8:(h + 1) * 8] = e8
        coeff_ref[:, h * 8:(h + 1) * 8] = cf


def _routing(x2, W_q, keysT, interpret=False):
    n_tok = x2.shape[0]
    grid = (n_tok // TB,)
    return pl.pallas_call(
        _routing_body,
        grid=grid,
        in_specs=[
            pl.BlockSpec((TB, DIM), lambda i: (i, 0)),
            pl.BlockSpec((DIM, 2 * HEADS * DIM_KEY), lambda i: (0, 0)),
            pl.BlockSpec((2, HEADS, DIM_KEY, NUM_KEYS), lambda i: (0, 0, 0, 0)),
        ],
        out_specs=[
            pl.BlockSpec((TB, KTOT), lambda i: (i, 0)),
            pl.BlockSpec((TB, KTOT), lambda i: (i, 0)),
        ],
        out_shape=[
            jax.ShapeDtypeStruct((n_tok, KTOT), jnp.int32),
            jax.ShapeDtypeStruct((n_tok, KTOT), jnp.float32),
        ],
        interpret=interpret,
    )(x2, W_q, keysT)


def _gelu16(v):
    z = jnp.abs(v) * jnp.float32(0.7071067811865476)
    t = jnp.float32(1.0) / (jnp.float32(1.0) + jnp.float32(0.3275911) * z)
    poly = t * (jnp.float32(0.254829592)
                + t * (jnp.float32(-0.284496736)
                       + t * (jnp.float32(1.421413741)
                              + t * (jnp.float32(-1.453152027)
                                     + t * jnp.float32(1.061405429)))))
    erf_abs = jnp.float32(1.0) - poly * jnp.exp(-z * z)
    erf = jnp.where(v >= jnp.float32(0.0), erf_abs, -erf_abs)
    return v * jnp.float32(0.5) * (jnp.float32(1.0) + erf)


_NSUB = 32
_NCH = DIM // 16
_HALF = KTOT // 2


def _make_sc_body(tpw):
  P = 4 * tpw

  def _sc_body(x_hbm, eidx_hbm, coeff_hbm, wd_hbm, wu_hbm, out_hbm,
               idx_all, cf_all, x8, out8, rows3, cg_v, sem0, sem1, sem2):
    wid = lax.axis_index("s") * 2 + lax.axis_index("c")
    iota16 = lax.iota(jnp.int32, 16)
    base_t = wid * tpw
    sems = (sem0, sem1, sem2)

    def issue(p):
        i = lax.div(p, 4)
        q = lax.rem(p, 4)
        half = lax.rem(q, 2)
        slot = lax.rem(p, 3)
        hq = _HALF // 2
        hs0 = pl.ds(pl.multiple_of(half * _HALF, 8), hq)
        hs1 = pl.ds(pl.multiple_of(half * _HALF + hq, 8), hq)
        for s in range(3):
            @pl.when(slot == s)
            def _():
                dst0 = rows3.at[pl.ds(s * _HALF, hq)]
                dst1 = rows3.at[pl.ds(s * _HALF + hq, hq)]

                @pl.when(q < 2)
                def _():
                    pltpu.async_copy(wd_hbm.at[idx_all.at[i, hs0]], dst0,
                                     sems[s])
                    pltpu.async_copy(wd_hbm.at[idx_all.at[i, hs1]], dst1,
                                     sems[s])

                @pl.when(q >= 2)
                def _():
                    pltpu.async_copy(wu_hbm.at[idx_all.at[i, hs0]], dst0,
                                     sems[s])
                    pltpu.async_copy(wu_hbm.at[idx_all.at[i, hs1]], dst1,
                                     sems[s])

    def wait_phase(p):
        slot = lax.rem(p, 3)
        for s in range(3):
            @pl.when(slot == s)
            def _():
                pltpu.make_async_copy(
                    wd_hbm.at[idx_all.at[0, pl.ds(0, _HALF)]],
                    rows3.at[pl.ds(s * _HALF, _HALF)], sems[s]).wait()

    def dots_half(i, xloc, half, rb):
        for c in range(2):
            def g2_body(g2, hv):
                jb = rb + c * 16 + g2 * 8

                def d_body(d, accs):
                    accs = list(accs)
                    for u in range(8):
                        sl = pl.ds(pl.multiple_of(d * 128 + u * 16, 16), 16)
                        xv = x8[xloc, sl]
                        for q in range(8):
                            accs[q] = accs[q] + rows3[jb + q, sl] * xv
                    return tuple(accs)

                z = jnp.zeros((16,), jnp.float32)
                accs = plsc.parallel_loop(0, _NCH // 8, 1,
                                          carry=(z,) * 8)(d_body)
                for q in range(8):
                    hv = jnp.where(iota16 == g2 * 8 + q, jnp.sum(accs[q]), hv)
                return hv

            hv = lax.fori_loop(0, 2, g2_body, jnp.zeros((16,), jnp.float32))
            ch = pl.multiple_of((half * 2 + c) * 16, 16)
            cg_v[pl.ds(ch, 16)] = (cf_all[i, pl.ds(ch, 16)] * _gelu16(hv))

    def acc_half(i, xloc, half, rb):
        dn = lax.GatherDimensionNumbers(
            offset_dims=(), collapsed_slice_dims=(0,), start_index_map=(0,))
        for cc in range(2):
            ch = pl.multiple_of((half * 2 + cc) * 16, 16)
            cvec = cg_v[pl.ds(ch, 16)]
            cbs = [lax.gather(cvec, jnp.full((16, 1), j, jnp.int32), dn,
                              slice_sizes=(1,),
                              mode=lax.GatherScatterMode.PROMISE_IN_BOUNDS)
                   for j in range(16)]
            jb = rb + cc * 16

            def d_body(d):
                sl = pl.ds(pl.multiple_of(d * 16, 16), 16)
                acc = out8[xloc, sl]
                for j in range(16):
                    acc = acc + rows3[jb + j, sl] * cbs[j]
                out8[xloc, sl] = acc

            plsc.parallel_loop(0, _NCH, 1, unroll=2)(d_body)

    pltpu.sync_copy(eidx_hbm.at[pl.ds(base_t, tpw)], idx_all)
    pltpu.sync_copy(coeff_hbm.at[pl.ds(base_t, tpw)], cf_all)
    pltpu.sync_copy(x_hbm.at[pl.ds(base_t, 8)], x8)
    issue(0)
    issue(1)

    def phase_body(p, carry):
        i = lax.div(p, 4)
        q = lax.rem(p, 4)
        xloc = lax.rem(i, 8)
        rb = lax.rem(p, 3) * _HALF

        @pl.when(jnp.logical_and(q == 0,
                                 jnp.logical_and(xloc == 0, i > 0)))
        def _():
            pltpu.sync_copy(
                x_hbm.at[pl.ds(pl.multiple_of(base_t + i, 8), 8)], x8)

        @pl.when(q == 0)
        def _():
            def z_body(d, c2):
                for u in range(8):
                    sl = pl.ds(pl.multiple_of(d * 128 + u * 16, 16), 16)
                    out8[xloc, sl] = jnp.zeros((16,), jnp.float32)
                return c2

            lax.fori_loop(0, _NCH // 8, z_body, 0)

        @pl.when(p < P - 2)
        def _():
            issue(p + 2)

        wait_phase(p)

        @pl.when(q < 2)
        def _():
            dots_half(i, xloc, lax.rem(q, 2), rb)

        @pl.when(q >= 2)
        def _():
            acc_half(i, xloc, lax.rem(q, 2), rb)

        @pl.when(jnp.logical_and(q == 3, xloc == 7))
        def _():
            pltpu.sync_copy(
                out8, out_hbm.at[pl.ds(pl.multiple_of(base_t + i - 7, 8), 8)])

        return carry

    lax.fori_loop(0, P, phase_body, 0)
  return _sc_body


def _sc_moe(x2, eidx, coeff, weight_down, weight_up):
    n_tok = x2.shape[0]
    tpw = n_tok // _NSUB
    mesh = plsc.VectorSubcoreMesh(core_axis_name="c", subcore_axis_name="s")
    f = functools.partial(
        pl.kernel,
        mesh=mesh,
        compiler_params=pltpu.CompilerParams(needs_layout_passes=False, use_tc_tiling_on_sc=False),
        out_type=jax.ShapeDtypeStruct((n_tok, DIM), jnp.float32),
        scratch_types=[
            pltpu.VMEM((tpw, KTOT), jnp.int32),
            pltpu.VMEM((tpw, KTOT), jnp.float32),
            pltpu.VMEM((8, DIM), jnp.float32),
            pltpu.VMEM((8, DIM), jnp.float32),
            pltpu.VMEM((3 * _HALF, DIM), jnp.float32),
            pltpu.VMEM((KTOT,), jnp.float32),
            pltpu.SemaphoreType.DMA,
            pltpu.SemaphoreType.DMA,
            pltpu.SemaphoreType.DMA,
        ],
    )(_make_sc_body(tpw))
    return f(x2, eidx, coeff, weight_down, weight_up)


N_CHUNKS = 8


def kernel(x, W_q, keys, weight_down, weight_up):
    b, n, d = x.shape
    x2 = x.reshape(n, d)
    keysT = jnp.transpose(keys, (2, 0, 3, 1))
    cs = n // N_CHUNKS
    outs = []
    for c in range(N_CHUNKS):
        xc = lax.slice_in_dim(x2, c * cs, (c + 1) * cs, axis=0)
        eidx, coeff = _routing(xc, W_q, keysT)
        outs.append(_sc_moe(xc, eidx, coeff, weight_down, weight_up))
    return jnp.concatenate(outs, axis=0).reshape(b, n, d)


# speedup vs baseline: 1.5772x; 1.5772x over previous
import functools

import jax
import jax.numpy as jnp
from jax import lax
from jax.experimental import pallas as pl
from jax.experimental.pallas import tpu as pltpu
from jax.experimental.pallas import tpu_sc as plsc

DIM = 1024
HEADS = 8
NUM_KEYS = 256
DIM_KEY = 128
TOPK = 8
KTOT = HEADS * TOPK
N_TOKENS = 2048
TB = 256
NEG = -3.0e38


def _top8(s, width):
    iota = lax.broadcasted_iota(jnp.int32, s.shape, 1).astype(jnp.float32)
    vs, ix = [], []
    for _ in range(8):
        m = jnp.max(s, axis=1, keepdims=True)
        pos = -jnp.max(jnp.where(s >= m, -iota, NEG), axis=1, keepdims=True)
        vs.append(m)
        ix.append(pos)
        s = jnp.where(iota == pos, NEG, s)
    return jnp.concatenate(vs, axis=1), jnp.concatenate(ix, axis=1)


def _routing_body(x_ref, wq_ref, kt_ref, eidx_ref, coeff_ref):
    xb = x_ref[...]
    for h in range(HEADS):
        svals, sidxs = [], []
        for p in range(2):
            col = (p * HEADS + h) * DIM_KEY
            q_ph = jnp.dot(xb, wq_ref[:, col:col + DIM_KEY],
                           preferred_element_type=jnp.float32)
            sim = jnp.dot(q_ph, kt_ref[p, h],
                          preferred_element_type=jnp.float32)
            v, i = _top8(sim, NUM_KEYS)
            svals.append(v)
            sidxs.append(i)
        sx, sy = svals
        ixx, ixy = sidxs
        alls = jnp.concatenate([sx[:, i:i + 1] + sy for i in range(8)], axis=1)
        alli = jnp.concatenate(
            [ixx[:, i:i + 1] * NUM_KEYS + ixy for i in range(8)], axis=1)
        iota = lax.broadcasted_iota(jnp.int32, alls.shape, 1).astype(jnp.float32)
        s = alls
        vs, es = [], []
        for _ in range(8):
            m = jnp.max(s, axis=1, keepdims=True)
            pos = -jnp.max(jnp.where(s >= m, -iota, NEG), axis=1,
                           keepdims=True)
            sel = iota == pos
            e = jnp.max(jnp.where(sel, alli, NEG), axis=1, keepdims=True)
            vs.append(m)
            es.append(e)
            s = jnp.where(sel, NEG, s)
        v8 = jnp.concatenate(vs, axis=1)
        e8 = jnp.concatenate(es, axis=1).astype(jnp.int32)
        m8 = jnp.max(v8, axis=1, keepdims=True)
        ex = jnp.exp(v8 - m8)
        cf = ex / jnp.sum(ex, axis=1, keepdims=True)
        eidx_ref[:, h * 8:(h + 1) * 8] = e8
        coeff_ref[:, h * 8:(h + 1) * 8] = cf


def _routing(x2, W_q, keysT, interpret=False):
    n_tok = x2.shape[0]
    grid = (n_tok // TB,)
    return pl.pallas_call(
        _routing_body,
        grid=grid,
        in_specs=[
            pl.BlockSpec((TB, DIM), lambda i: (i, 0)),
            pl.BlockSpec((DIM, 2 * HEADS * DIM_KEY), lambda i: (0, 0)),
            pl.BlockSpec((2, HEADS, DIM_KEY, NUM_KEYS), lambda i: (0, 0, 0, 0)),
        ],
        out_specs=[
            pl.BlockSpec((TB, KTOT), lambda i: (i, 0)),
            pl.BlockSpec((TB, KTOT), lambda i: (i, 0)),
        ],
        out_shape=[
            jax.ShapeDtypeStruct((n_tok, KTOT), jnp.int32),
            jax.ShapeDtypeStruct((n_tok, KTOT), jnp.float32),
        ],
        interpret=interpret,
    )(x2, W_q, keysT)


def _gelu16(v):
    z = jnp.abs(v) * jnp.float32(0.7071067811865476)
    t = jnp.float32(1.0) / (jnp.float32(1.0) + jnp.float32(0.3275911) * z)
    poly = t * (jnp.float32(0.254829592)
                + t * (jnp.float32(-0.284496736)
                       + t * (jnp.float32(1.421413741)
                              + t * (jnp.float32(-1.453152027)
                                     + t * jnp.float32(1.061405429)))))
    erf_abs = jnp.float32(1.0) - poly * jnp.exp(-z * z)
    erf = jnp.where(v >= jnp.float32(0.0), erf_abs, -erf_abs)
    return v * jnp.float32(0.5) * (jnp.float32(1.0) + erf)


_NSUB = 32
_NCH = DIM // 16
_HALF = KTOT // 2


def _make_sc_body(tpw):
  P = 4 * tpw

  def _sc_body(x_hbm, eidx_hbm, coeff_hbm, wd_hbm, wu_hbm, out_hbm,
               idx_all, cf_all, x8, out8, rows3, cg_v, sem0, sem1, sem2):
    wid = lax.axis_index("s") * 2 + lax.axis_index("c")
    iota16 = lax.iota(jnp.int32, 16)
    base_t = wid * tpw
    sems = (sem0, sem1, sem2)

    def issue(p):
        i = lax.div(p, 4)
        q = lax.rem(p, 4)
        half = lax.rem(q, 2)
        slot = lax.rem(p, 3)
        hq = _HALF // 2
        hs0 = pl.ds(pl.multiple_of(half * _HALF, 8), hq)
        hs1 = pl.ds(pl.multiple_of(half * _HALF + hq, 8), hq)
        for s in range(3):
            @pl.when(slot == s)
            def _():
                dst0 = rows3.at[pl.ds(s * _HALF, hq)]
                dst1 = rows3.at[pl.ds(s * _HALF + hq, hq)]

                @pl.when(q < 2)
                def _():
                    pltpu.async_copy(wd_hbm.at[idx_all.at[i, hs0]], dst0,
                                     sems[s])
                    pltpu.async_copy(wd_hbm.at[idx_all.at[i, hs1]], dst1,
                                     sems[s])

                @pl.when(q >= 2)
                def _():
                    pltpu.async_copy(wu_hbm.at[idx_all.at[i, hs0]], dst0,
                                     sems[s])
                    pltpu.async_copy(wu_hbm.at[idx_all.at[i, hs1]], dst1,
                                     sems[s])

    def wait_phase(p):
        slot = lax.rem(p, 3)
        for s in range(3):
            @pl.when(slot == s)
            def _():
                pltpu.make_async_copy(
                    wd_hbm.at[idx_all.at[0, pl.ds(0, _HALF)]],
                    rows3.at[pl.ds(s * _HALF, _HALF)], sems[s]).wait()

    def dots_half(i, xloc, half, rb):
        for c in range(2):
            def g2_body(g2, hv):
                jb = rb + c * 16 + g2 * 8

                def d_body(d, accs):
                    accs = list(accs)
                    for u in range(8):
                        sl = pl.ds(pl.multiple_of(d * 128 + u * 16, 16), 16)
                        xv = x8[xloc, sl]
                        for q in range(8):
                            accs[q] = accs[q] + rows3[jb + q, sl] * xv
                    return tuple(accs)

                z = jnp.zeros((16,), jnp.float32)
                accs = plsc.parallel_loop(0, _NCH // 8, 1,
                                          carry=(z,) * 8)(d_body)
                for q in range(8):
                    hv = jnp.where(iota16 == g2 * 8 + q, jnp.sum(accs[q]), hv)
                return hv

            hv = lax.fori_loop(0, 2, g2_body, jnp.zeros((16,), jnp.float32))
            ch = pl.multiple_of((half * 2 + c) * 16, 16)
            cg_v[pl.ds(ch, 16)] = (cf_all[i, pl.ds(ch, 16)] * _gelu16(hv))

    def acc_half(i, xloc, half, rb):
        dn = lax.GatherDimensionNumbers(
            offset_dims=(), collapsed_slice_dims=(0,), start_index_map=(0,))
        for cc in range(2):
            ch = pl.multiple_of((half * 2 + cc) * 16, 16)
            cvec = cg_v[pl.ds(ch, 16)]
            cbs = [lax.gather(cvec, jnp.full((16, 1), j, jnp.int32), dn,
                              slice_sizes=(1,),
                              mode=lax.GatherScatterMode.PROMISE_IN_BOUNDS)
                   for j in range(16)]
            jb = rb + cc * 16

            def d_body(d):
                sl = pl.ds(pl.multiple_of(d * 16, 16), 16)
                acc = out8[xloc, sl]
                for j in range(16):
                    acc = acc + rows3[jb + j, sl] * cbs[j]
                out8[xloc, sl] = acc

            plsc.parallel_loop(0, _NCH, 1, unroll=2)(d_body)

    pltpu.sync_copy(eidx_hbm.at[pl.ds(base_t, tpw)], idx_all)
    pltpu.sync_copy(coeff_hbm.at[pl.ds(base_t, tpw)], cf_all)
    pltpu.sync_copy(x_hbm.at[pl.ds(base_t, 8)], x8)
    issue(0)
    issue(1)

    def phase_body(p, carry):
        i = lax.div(p, 4)
        q = lax.rem(p, 4)
        xloc = lax.rem(i, 8)
        rb = lax.rem(p, 3) * _HALF

        @pl.when(jnp.logical_and(q == 0,
                                 jnp.logical_and(xloc == 0, i > 0)))
        def _():
            pltpu.sync_copy(
                x_hbm.at[pl.ds(pl.multiple_of(base_t + i, 8), 8)], x8)

        @pl.when(q == 0)
        def _():
            def z_body(d, c2):
                for u in range(8):
                    sl = pl.ds(pl.multiple_of(d * 128 + u * 16, 16), 16)
                    out8[xloc, sl] = jnp.zeros((16,), jnp.float32)
                return c2

            lax.fori_loop(0, _NCH // 8, z_body, 0)

        @pl.when(p < P - 2)
        def _():
            issue(p + 2)

        wait_phase(p)

        @pl.when(q < 2)
        def _():
            dots_half(i, xloc, lax.rem(q, 2), rb)

        @pl.when(q >= 2)
        def _():
            acc_half(i, xloc, lax.rem(q, 2), rb)

        @pl.when(jnp.logical_and(q == 3, xloc == 7))
        def _():
            pltpu.sync_copy(
                out8, out_hbm.at[pl.ds(pl.multiple_of(base_t + i - 7, 8), 8)])

        return carry

    lax.fori_loop(0, P, phase_body, 0)
  return _sc_body


def _sc_moe(x2, eidx, coeff, weight_down, weight_up):
    n_tok = x2.shape[0]
    tpw = n_tok // _NSUB
    mesh = plsc.VectorSubcoreMesh(core_axis_name="c", subcore_axis_name="s")
    f = functools.partial(
        pl.kernel,
        mesh=mesh,
        compiler_params=pltpu.CompilerParams(needs_layout_passes=False),
        out_type=jax.ShapeDtypeStruct((n_tok, DIM), jnp.float32),
        scratch_types=[
            pltpu.VMEM((tpw, KTOT), jnp.int32),
            pltpu.VMEM((tpw, KTOT), jnp.float32),
            pltpu.VMEM((8, DIM), jnp.float32),
            pltpu.VMEM((8, DIM), jnp.float32),
            pltpu.VMEM((3 * _HALF, DIM), jnp.float32),
            pltpu.VMEM((KTOT,), jnp.float32),
            pltpu.SemaphoreType.DMA,
            pltpu.SemaphoreType.DMA,
            pltpu.SemaphoreType.DMA,
        ],
    )(_make_sc_body(tpw))
    return f(x2, eidx, coeff, weight_down, weight_up)


N_CHUNKS = 8


def kernel(x, W_q, keys, weight_down, weight_up):
    b, n, d = x.shape
    x2 = x.reshape(n, d)
    keysT = jnp.transpose(keys, (2, 0, 3, 1))
    cs = n // N_CHUNKS
    outs = []
    for c in range(N_CHUNKS):
        xc = lax.slice_in_dim(x2, c * cs, (c + 1) * cs, axis=0)
        eidx, coeff = _routing(xc, W_q, keysT)
        outs.append(_sc_moe(xc, eidx, coeff, weight_down, weight_up))
    return jnp.concatenate(outs, axis=0).reshape(b, n, d)
